# baseline (speedup 1.0000x reference)
import functools
import jax
import jax.numpy as jnp
from jax import lax
from jax.experimental import pallas as pl
from jax.experimental.pallas import tpu as pltpu
from jax.experimental.pallas import tpu_sc as plsc

_F_PAD = 256
_N_PAD = 20480
_TILE = 2048
_HI = jax.lax.Precision.HIGHEST


def _rot6_cols(sl):
    a1 = sl[3:6]
    a2 = sl[6:9]
    n1 = jnp.maximum(jnp.sqrt(a1[0] * a1[0] + a1[1] * a1[1] + a1[2] * a1[2]), 1e-8)
    b1 = [a1[0] / n1, a1[1] / n1, a1[2] / n1]
    d = b1[0] * a2[0] + b1[1] * a2[1] + b1[2] * a2[2]
    c2 = [a2[0] - d * b1[0], a2[1] - d * b1[1], a2[2] - d * b1[2]]
    n2 = jnp.maximum(jnp.sqrt(c2[0] * c2[0] + c2[1] * c2[1] + c2[2] * c2[2]), 1e-8)
    b2 = [c2[0] / n2, c2[1] / n2, c2[2] / n2]
    b3 = [b1[1] * b2[2] - b1[2] * b2[1],
          b1[2] * b2[0] - b1[0] * b2[2],
          b1[0] * b2[1] - b1[1] * b2[0]]
    return [b1, b2, b3]


def _softmax_cols(x):
    m = jnp.max(x, axis=0, keepdims=True)
    e = jnp.exp(x - m)
    return e / jnp.sum(e, axis=0, keepdims=True)


def _prep_kernel(ts_ref, d1_ref, d0_ref, mc0t_ref, mallt_ref, p0t_ref):
    tsb = jnp.broadcast_to(ts_ref[:, :], (16, _F_PAD))
    fio = jax.lax.broadcasted_iota(jnp.int32, (16, _F_PAD), 1)
    oht = (tsb == fio).astype(jnp.float32)

    for c in range(9):
        mallt_ref[c * 16:(c + 1) * 16, :] = jnp.dot(
            oht, d1_ref[c], preferred_element_type=jnp.float32, precision=_HI)

    sm0t = _softmax_cols(mc0t_ref[:, :])
    sl0 = []
    for c in range(9):
        g0t = jnp.dot(oht, d0_ref[c], preferred_element_type=jnp.float32,
                      precision=_HI)
        sl0.append(jnp.dot(g0t, sm0t, preferred_element_type=jnp.float32,
                           precision=_HI))
    b0 = _rot6_cols(sl0)
    for k in range(3):
        for j in range(3):
            p0t_ref[(k * 4 + j) * 16:(k * 4 + j + 1) * 16, :] = b0[j][k]
        p0t_ref[(k * 4 + 3) * 16:(k * 4 + 4) * 16, :] = sl0[k]


def _make_sc_gather(n_rows, d, n_chunks=2):
    info = plsc.get_sparse_core_info()
    nw = info.num_cores * info.num_subcores
    b_per_w = n_rows // nw
    chunk = b_per_w // n_chunks
    mesh = plsc.VectorSubcoreMesh(core_axis_name="c", subcore_axis_name="s")

    @functools.partial(
        pl.kernel, mesh=mesh,
        out_type=jax.ShapeDtypeStruct((n_rows, d), jnp.float32),
        scratch_types=[
            pltpu.VMEM((chunk,), jnp.int32),
            pltpu.VMEM((chunk, d), jnp.float32),
            pltpu.SemaphoreType.DMA,
        ],
    )
    def k(table_hbm, idx_hbm, out_hbm, idx_v, rows_v, sem):
        wid = lax.axis_index("s") * info.num_cores + lax.axis_index("c")
        for ch in range(n_chunks):
            base = wid * b_per_w + ch * chunk
            pltpu.sync_copy(idx_hbm.at[pl.ds(base, chunk)], idx_v)
            pltpu.async_copy(table_hbm.at[idx_v], rows_v, sem).wait()
            pltpu.sync_copy(rows_v, out_hbm.at[pl.ds(base, chunk)])

    return k


def _main_kernel(coefst_ref, part_ref, mallt_ref, pg_ref, out_ref):
    L = coefst_ref.shape[1]
    smt = _softmax_cols(coefst_ref[:, :])
    p = part_ref[:, :]

    sub32 = jax.lax.broadcasted_iota(jnp.int32, (32, L), 0)
    oh32 = (sub32 == jnp.broadcast_to(p, (32, L))).astype(jnp.float32)

    pm = jnp.broadcast_to(oh32[:, None, :], (32, 16, L)).reshape(512, L)
    smtile = jnp.broadcast_to(smt[None, :, :], (32, 16, L)).reshape(512, L)
    at = pm * smtile
    out9t = jnp.dot(mallt_ref[:, :], at, preferred_element_type=jnp.float32,
                    precision=_HI)

    p0gt = jnp.transpose(pg_ref[:, :])[:192]

    sl = [out9t[c * 16:(c + 1) * 16, :] for c in range(9)]
    b = _rot6_cols(sl)
    res = []
    for i in range(3):
        for j in range(4):
            acc = b[0][i] * p0gt[(0 * 4 + j) * 16:(0 * 4 + j + 1) * 16, :]
            acc += b[1][i] * p0gt[(1 * 4 + j) * 16:(1 * 4 + j + 1) * 16, :]
            acc += b[2][i] * p0gt[(2 * 4 + j) * 16:(2 * 4 + j + 1) * 16, :]
            if j == 3:
                acc += sl[i]
            res.append(acc)
    arr = jnp.stack(res, axis=1).reshape(192, L)
    out_ref[:, :] = jnp.transpose(arr)


def kernel(rots_l0, transls_l0, motion_coefs_l0, rots_l1, transls_l1,
           motion_coefs_l1, parent_indices_l0, parent_indices_l1, ts):
    N0, B, Fr = rots_l1.shape[0], rots_l1.shape[1], rots_l1.shape[2]
    T = ts.shape[0]
    N1 = motion_coefs_l1.shape[0]

    d1 = jnp.concatenate([transls_l1, rots_l1], axis=-1)
    d1 = jnp.transpose(d1, (3, 2, 0, 1)).reshape(9, Fr, N0 * B)
    d1 = jnp.pad(d1, ((0, 0), (0, _F_PAD - Fr), (0, 0)))
    d0 = jnp.concatenate([transls_l0, rots_l0], axis=-1)[0]
    d0 = jnp.transpose(d0, (2, 1, 0))
    d0 = jnp.pad(d0, ((0, 0), (0, _F_PAD - Fr), (0, 0)))
    ts2 = ts.reshape(T, 1).astype(jnp.int32)
    mc0t = motion_coefs_l0.T

    mallt, p0t = pl.pallas_call(
        _prep_kernel,
        out_shape=[
            jax.ShapeDtypeStruct((9 * 16, N0 * B), jnp.float32),
            jax.ShapeDtypeStruct((12 * 16, N0), jnp.float32),
        ],
    )(ts2, d1, d0, mc0t)

    coefst = motion_coefs_l1.T
    part = parent_indices_l1.astype(jnp.int32).reshape(1, N1)

    n_sc = _N_PAD
    idx_sc = jnp.pad(parent_indices_l1.astype(jnp.int32), (0, n_sc - N1))
    table = jnp.pad(p0t.T, ((0, 0), (0, 256 - 12 * 16)))
    pg = _make_sc_gather(n_sc, 256)(table, idx_sc)

    grid = (N1 + _TILE - 1) // _TILE
    out = pl.pallas_call(
        _main_kernel,
        grid=(grid,),
        in_specs=[
            pl.BlockSpec((16, _TILE), lambda i: (0, i)),
            pl.BlockSpec((1, _TILE), lambda i: (0, i)),
            pl.BlockSpec((9 * 16, N0 * B), lambda i: (0, 0)),
            pl.BlockSpec((_TILE, 256), lambda i: (i, 0)),
        ],
        out_specs=pl.BlockSpec((_TILE, 12 * 16), lambda i: (i, 0)),
        out_shape=jax.ShapeDtypeStruct((N1, 12 * 16), jnp.float32),
    )(coefst, part, mallt, pg)

    return out.reshape(N1, T, 3, 4)


# speedup vs baseline: 1.9900x; 1.9900x over previous
import jax
import jax.numpy as jnp
from jax.experimental import pallas as pl

_F_PAD = 256
_N_PAD = 20480
_TILE = 2048
_HI = jax.lax.Precision.HIGHEST


def _rot6_cols(sl):
    a1 = sl[3:6]
    a2 = sl[6:9]
    n1 = jnp.maximum(jnp.sqrt(a1[0] * a1[0] + a1[1] * a1[1] + a1[2] * a1[2]), 1e-8)
    b1 = [a1[0] / n1, a1[1] / n1, a1[2] / n1]
    d = b1[0] * a2[0] + b1[1] * a2[1] + b1[2] * a2[2]
    c2 = [a2[0] - d * b1[0], a2[1] - d * b1[1], a2[2] - d * b1[2]]
    n2 = jnp.maximum(jnp.sqrt(c2[0] * c2[0] + c2[1] * c2[1] + c2[2] * c2[2]), 1e-8)
    b2 = [c2[0] / n2, c2[1] / n2, c2[2] / n2]
    b3 = [b1[1] * b2[2] - b1[2] * b2[1],
          b1[2] * b2[0] - b1[0] * b2[2],
          b1[0] * b2[1] - b1[1] * b2[0]]
    return [b1, b2, b3]


def _softmax_cols(x):
    m = jnp.max(x, axis=0, keepdims=True)
    e = jnp.exp(x - m)
    return e / jnp.sum(e, axis=0, keepdims=True)


def _prep_kernel(ts_ref, d1_ref, d0_ref, mc0t_ref, mallt_ref, p0t_ref):
    tsb = jnp.broadcast_to(ts_ref[:, :], (16, _F_PAD))
    fio = jax.lax.broadcasted_iota(jnp.int32, (16, _F_PAD), 1)
    oht = (tsb == fio).astype(jnp.float32)

    for c in range(9):
        mallt_ref[c * 16:(c + 1) * 16, :] = jnp.dot(
            oht, d1_ref[c], preferred_element_type=jnp.float32, precision=_HI)

    sm0t = _softmax_cols(mc0t_ref[:, :])
    sl0 = []
    for c in range(9):
        g0t = jnp.dot(oht, d0_ref[c], preferred_element_type=jnp.float32,
                      precision=_HI)
        sl0.append(jnp.dot(g0t, sm0t, preferred_element_type=jnp.float32,
                           precision=_HI))
    b0 = _rot6_cols(sl0)
    for k in range(3):
        for j in range(3):
            p0t_ref[(k * 4 + j) * 16:(k * 4 + j + 1) * 16, :] = b0[j][k]
        p0t_ref[(k * 4 + 3) * 16:(k * 4 + 4) * 16, :] = sl0[k]


def _main_kernel(coefst_ref, part_ref, mallt_ref, p0t_ref, out_ref):
    L = coefst_ref.shape[1]
    smt = _softmax_cols(coefst_ref[:, :])
    p = part_ref[:, :]

    sub32 = jax.lax.broadcasted_iota(jnp.int32, (32, L), 0)
    oh32 = (sub32 == jnp.broadcast_to(p, (32, L))).astype(jnp.float32)

    smt_hi = smt.astype(jnp.bfloat16)
    smt_lo = (smt - smt_hi.astype(jnp.float32)).astype(jnp.bfloat16)
    pm16 = jnp.broadcast_to(oh32[:, None, :], (32, 16, L)).reshape(512, L).astype(jnp.bfloat16)
    at_hi = pm16 * jnp.broadcast_to(smt_hi[None, :, :], (32, 16, L)).reshape(512, L)
    at_lo = pm16 * jnp.broadcast_to(smt_lo[None, :, :], (32, 16, L)).reshape(512, L)
    mall = mallt_ref[:, :]
    mall_hi = mall.astype(jnp.bfloat16)
    mall_lo = (mall - mall_hi.astype(jnp.float32)).astype(jnp.bfloat16)
    out9t = (jnp.dot(mall_hi, at_hi, preferred_element_type=jnp.float32)
             + jnp.dot(mall_hi, at_lo, preferred_element_type=jnp.float32)
             + jnp.dot(mall_lo, at_hi, preferred_element_type=jnp.float32))

    p0gt = jnp.dot(p0t_ref[:, :], oh32, preferred_element_type=jnp.float32,
                   precision=_HI)

    sl = [out9t[c * 16:(c + 1) * 16, :] for c in range(9)]
    b = _rot6_cols(sl)
    res = []
    for i in range(3):
        for j in range(4):
            acc = b[0][i] * p0gt[(0 * 4 + j) * 16:(0 * 4 + j + 1) * 16, :]
            acc += b[1][i] * p0gt[(1 * 4 + j) * 16:(1 * 4 + j + 1) * 16, :]
            acc += b[2][i] * p0gt[(2 * 4 + j) * 16:(2 * 4 + j + 1) * 16, :]
            if j == 3:
                acc += sl[i]
            res.append(acc)
    arr = jnp.stack(res, axis=1).reshape(192, L)
    out_ref[:, :] = jnp.transpose(arr)


def kernel(rots_l0, transls_l0, motion_coefs_l0, rots_l1, transls_l1,
           motion_coefs_l1, parent_indices_l0, parent_indices_l1, ts):
    N0, B, Fr = rots_l1.shape[0], rots_l1.shape[1], rots_l1.shape[2]
    T = ts.shape[0]
    N1 = motion_coefs_l1.shape[0]

    d1 = jnp.concatenate([transls_l1, rots_l1], axis=-1)
    d1 = jnp.transpose(d1, (3, 2, 0, 1)).reshape(9, Fr, N0 * B)
    d1 = jnp.pad(d1, ((0, 0), (0, _F_PAD - Fr), (0, 0)))
    d0 = jnp.concatenate([transls_l0, rots_l0], axis=-1)[0]
    d0 = jnp.transpose(d0, (2, 1, 0))
    d0 = jnp.pad(d0, ((0, 0), (0, _F_PAD - Fr), (0, 0)))
    ts2 = ts.reshape(T, 1).astype(jnp.int32)
    mc0t = motion_coefs_l0.T

    mallt, p0t = pl.pallas_call(
        _prep_kernel,
        out_shape=[
            jax.ShapeDtypeStruct((9 * 16, N0 * B), jnp.float32),
            jax.ShapeDtypeStruct((12 * 16, N0), jnp.float32),
        ],
    )(ts2, d1, d0, mc0t)

    coefst = motion_coefs_l1.T
    part = parent_indices_l1.astype(jnp.int32).reshape(1, N1)
    grid = (N1 + _TILE - 1) // _TILE
    out = pl.pallas_call(
        _main_kernel,
        grid=(grid,),
        in_specs=[
            pl.BlockSpec((16, _TILE), lambda i: (0, i)),
            pl.BlockSpec((1, _TILE), lambda i: (0, i)),
            pl.BlockSpec((9 * 16, N0 * B), lambda i: (0, 0)),
            pl.BlockSpec((12 * 16, N0), lambda i: (0, 0)),
        ],
        out_specs=pl.BlockSpec((_TILE, 12 * 16), lambda i: (i, 0)),
        out_shape=jax.ShapeDtypeStruct((N1, 12 * 16), jnp.float32),
    )(coefst, part, mallt, p0t)

    return out.reshape(N1, T, 3, 4)


# speedup vs baseline: 2.1081x; 1.0593x over previous
import jax
import jax.numpy as jnp
from jax.experimental import pallas as pl

_F_PAD = 256
_N_PAD = 20480
_TILE = 2048
_HI = jax.lax.Precision.HIGHEST


def _rot6_cols(sl):
    a1 = sl[3:6]
    a2 = sl[6:9]
    n1 = jnp.maximum(jnp.sqrt(a1[0] * a1[0] + a1[1] * a1[1] + a1[2] * a1[2]), 1e-8)
    b1 = [a1[0] / n1, a1[1] / n1, a1[2] / n1]
    d = b1[0] * a2[0] + b1[1] * a2[1] + b1[2] * a2[2]
    c2 = [a2[0] - d * b1[0], a2[1] - d * b1[1], a2[2] - d * b1[2]]
    n2 = jnp.maximum(jnp.sqrt(c2[0] * c2[0] + c2[1] * c2[1] + c2[2] * c2[2]), 1e-8)
    b2 = [c2[0] / n2, c2[1] / n2, c2[2] / n2]
    b3 = [b1[1] * b2[2] - b1[2] * b2[1],
          b1[2] * b2[0] - b1[0] * b2[2],
          b1[0] * b2[1] - b1[1] * b2[0]]
    return [b1, b2, b3]


def _softmax_cols(x):
    m = jnp.max(x, axis=0, keepdims=True)
    e = jnp.exp(x - m)
    return e / jnp.sum(e, axis=0, keepdims=True)


def _prep_kernel(ts_ref, d1_ref, d0_ref, mc0t_ref, mallt_ref, p0t_ref):
    tsb = jnp.broadcast_to(ts_ref[:, :], (16, _F_PAD))
    fio = jax.lax.broadcasted_iota(jnp.int32, (16, _F_PAD), 1)
    oht = (tsb == fio).astype(jnp.float32)

    for c in range(9):
        mallt_ref[c * 16:(c + 1) * 16, :] = jnp.dot(
            oht, d1_ref[c], preferred_element_type=jnp.float32, precision=_HI)

    sm0t = _softmax_cols(mc0t_ref[:, :])
    sl0 = []
    for c in range(9):
        g0t = jnp.dot(oht, d0_ref[c], preferred_element_type=jnp.float32,
                      precision=_HI)
        sl0.append(jnp.dot(g0t, sm0t, preferred_element_type=jnp.float32,
                           precision=_HI))
    b0 = _rot6_cols(sl0)
    for k in range(3):
        for j in range(3):
            p0t_ref[(k * 4 + j) * 16:(k * 4 + j + 1) * 16, :] = b0[j][k]
        p0t_ref[(k * 4 + 3) * 16:(k * 4 + 4) * 16, :] = sl0[k]


def _main_kernel(coefst_ref, part_ref, mallt_ref, p0t_ref, out_ref):
    L = coefst_ref.shape[1]
    smt = _softmax_cols(coefst_ref[:, :])
    p = part_ref[:, :]

    sub32 = jax.lax.broadcasted_iota(jnp.int32, (32, L), 0)
    oh32 = (sub32 == jnp.broadcast_to(p, (32, L))).astype(jnp.float32)

    smt_hi = smt.astype(jnp.bfloat16)
    smt_lo = (smt - smt_hi.astype(jnp.float32)).astype(jnp.bfloat16)
    pm16 = jnp.broadcast_to(oh32[:, None, :], (32, 16, L)).reshape(512, L).astype(jnp.bfloat16)
    at_hi = pm16 * jnp.broadcast_to(smt_hi[None, :, :], (32, 16, L)).reshape(512, L)
    at_lo = pm16 * jnp.broadcast_to(smt_lo[None, :, :], (32, 16, L)).reshape(512, L)
    mall = mallt_ref[:, :]
    mall_hi = mall.astype(jnp.bfloat16)
    mall_lo = (mall - mall_hi.astype(jnp.float32)).astype(jnp.bfloat16)
    out9t = (jnp.dot(mall_hi, at_hi, preferred_element_type=jnp.float32)
             + jnp.dot(mall_hi, at_lo, preferred_element_type=jnp.float32)
             + jnp.dot(mall_lo, at_hi, preferred_element_type=jnp.float32))

    p0 = p0t_ref[:, :]
    p0_hi = p0.astype(jnp.bfloat16)
    p0_lo = (p0 - p0_hi.astype(jnp.float32)).astype(jnp.bfloat16)
    oh16 = oh32.astype(jnp.bfloat16)
    p0gt = (jnp.dot(p0_hi, oh16, preferred_element_type=jnp.float32)
            + jnp.dot(p0_lo, oh16, preferred_element_type=jnp.float32))

    sl = [out9t[c * 16:(c + 1) * 16, :] for c in range(9)]
    b = _rot6_cols(sl)
    res = []
    for i in range(3):
        for j in range(4):
            acc = b[0][i] * p0gt[(0 * 4 + j) * 16:(0 * 4 + j + 1) * 16, :]
            acc += b[1][i] * p0gt[(1 * 4 + j) * 16:(1 * 4 + j + 1) * 16, :]
            acc += b[2][i] * p0gt[(2 * 4 + j) * 16:(2 * 4 + j + 1) * 16, :]
            if j == 3:
                acc += sl[i]
            res.append(acc)
    arr = jnp.stack(res, axis=1).reshape(192, L)
    out_ref[:, :] = jnp.transpose(arr)


def kernel(rots_l0, transls_l0, motion_coefs_l0, rots_l1, transls_l1,
           motion_coefs_l1, parent_indices_l0, parent_indices_l1, ts):
    N0, B, Fr = rots_l1.shape[0], rots_l1.shape[1], rots_l1.shape[2]
    T = ts.shape[0]
    N1 = motion_coefs_l1.shape[0]

    d1 = jnp.concatenate([transls_l1, rots_l1], axis=-1)
    d1 = jnp.transpose(d1, (3, 2, 0, 1)).reshape(9, Fr, N0 * B)
    d1 = jnp.pad(d1, ((0, 0), (0, _F_PAD - Fr), (0, 0)))
    d0 = jnp.concatenate([transls_l0, rots_l0], axis=-1)[0]
    d0 = jnp.transpose(d0, (2, 1, 0))
    d0 = jnp.pad(d0, ((0, 0), (0, _F_PAD - Fr), (0, 0)))
    ts2 = ts.reshape(T, 1).astype(jnp.int32)
    mc0t = motion_coefs_l0.T

    mallt, p0t = pl.pallas_call(
        _prep_kernel,
        out_shape=[
            jax.ShapeDtypeStruct((9 * 16, N0 * B), jnp.float32),
            jax.ShapeDtypeStruct((12 * 16, N0), jnp.float32),
        ],
    )(ts2, d1, d0, mc0t)

    coefst = motion_coefs_l1.T
    part = parent_indices_l1.astype(jnp.int32).reshape(1, N1)
    grid = (N1 + _TILE - 1) // _TILE
    out = pl.pallas_call(
        _main_kernel,
        grid=(grid,),
        in_specs=[
            pl.BlockSpec((16, _TILE), lambda i: (0, i)),
            pl.BlockSpec((1, _TILE), lambda i: (0, i)),
            pl.BlockSpec((9 * 16, N0 * B), lambda i: (0, 0)),
            pl.BlockSpec((12 * 16, N0), lambda i: (0, 0)),
        ],
        out_specs=pl.BlockSpec((_TILE, 12 * 16), lambda i: (i, 0)),
        out_shape=jax.ShapeDtypeStruct((N1, 12 * 16), jnp.float32),
    )(coefst, part, mallt, p0t)

    return out.reshape(N1, T, 3, 4)


# speedup vs baseline: 2.1259x; 1.0084x over previous
import jax
import jax.numpy as jnp
from jax.experimental import pallas as pl

_F_PAD = 256
_N_PAD = 20480
_TILE = 4096
_HI = jax.lax.Precision.HIGHEST


def _rot6_cols(sl):
    a1 = sl[3:6]
    a2 = sl[6:9]
    n1 = jnp.maximum(jnp.sqrt(a1[0] * a1[0] + a1[1] * a1[1] + a1[2] * a1[2]), 1e-8)
    b1 = [a1[0] / n1, a1[1] / n1, a1[2] / n1]
    d = b1[0] * a2[0] + b1[1] * a2[1] + b1[2] * a2[2]
    c2 = [a2[0] - d * b1[0], a2[1] - d * b1[1], a2[2] - d * b1[2]]
    n2 = jnp.maximum(jnp.sqrt(c2[0] * c2[0] + c2[1] * c2[1] + c2[2] * c2[2]), 1e-8)
    b2 = [c2[0] / n2, c2[1] / n2, c2[2] / n2]
    b3 = [b1[1] * b2[2] - b1[2] * b2[1],
          b1[2] * b2[0] - b1[0] * b2[2],
          b1[0] * b2[1] - b1[1] * b2[0]]
    return [b1, b2, b3]


def _softmax_cols(x):
    m = jnp.max(x, axis=0, keepdims=True)
    e = jnp.exp(x - m)
    return e / jnp.sum(e, axis=0, keepdims=True)


def _prep_kernel(ts_ref, d1_ref, d0_ref, mc0t_ref, mallt_ref, p0t_ref):
    tsb = jnp.broadcast_to(ts_ref[:, :], (16, _F_PAD))
    fio = jax.lax.broadcasted_iota(jnp.int32, (16, _F_PAD), 1)
    oht = (tsb == fio).astype(jnp.float32)

    for c in range(9):
        mallt_ref[c * 16:(c + 1) * 16, :] = jnp.dot(
            oht, d1_ref[c], preferred_element_type=jnp.float32, precision=_HI)

    sm0t = _softmax_cols(mc0t_ref[:, :])
    sl0 = []
    for c in range(9):
        g0t = jnp.dot(oht, d0_ref[c], preferred_element_type=jnp.float32,
                      precision=_HI)
        sl0.append(jnp.dot(g0t, sm0t, preferred_element_type=jnp.float32,
                           precision=_HI))
    b0 = _rot6_cols(sl0)
    for k in range(3):
        for j in range(3):
            p0t_ref[(k * 4 + j) * 16:(k * 4 + j + 1) * 16, :] = b0[j][k]
        p0t_ref[(k * 4 + 3) * 16:(k * 4 + 4) * 16, :] = sl0[k]


def _main_kernel(coefst_ref, part_ref, mallt_ref, p0t_ref, out_ref):
    L = coefst_ref.shape[1]
    smt = _softmax_cols(coefst_ref[:, :])
    p = part_ref[:, :]

    sub32 = jax.lax.broadcasted_iota(jnp.int32, (32, L), 0)
    oh32 = (sub32 == jnp.broadcast_to(p, (32, L))).astype(jnp.float32)

    smt_hi = smt.astype(jnp.bfloat16)
    smt_lo = (smt - smt_hi.astype(jnp.float32)).astype(jnp.bfloat16)
    pm16 = jnp.broadcast_to(oh32[:, None, :], (32, 16, L)).reshape(512, L).astype(jnp.bfloat16)
    at_hi = pm16 * jnp.broadcast_to(smt_hi[None, :, :], (32, 16, L)).reshape(512, L)
    at_lo = pm16 * jnp.broadcast_to(smt_lo[None, :, :], (32, 16, L)).reshape(512, L)
    mall = mallt_ref[:, :]
    mall_hi = mall.astype(jnp.bfloat16)
    mall_lo = (mall - mall_hi.astype(jnp.float32)).astype(jnp.bfloat16)
    out9t = (jnp.dot(mall_hi, at_hi, preferred_element_type=jnp.float32)
             + jnp.dot(mall_hi, at_lo, preferred_element_type=jnp.float32)
             + jnp.dot(mall_lo, at_hi, preferred_element_type=jnp.float32))

    p0 = p0t_ref[:, :]
    p0_hi = p0.astype(jnp.bfloat16)
    p0_lo = (p0 - p0_hi.astype(jnp.float32)).astype(jnp.bfloat16)
    oh16 = oh32.astype(jnp.bfloat16)
    p0gt = (jnp.dot(p0_hi, oh16, preferred_element_type=jnp.float32)
            + jnp.dot(p0_lo, oh16, preferred_element_type=jnp.float32))

    sl = [out9t[c * 16:(c + 1) * 16, :] for c in range(9)]
    b = _rot6_cols(sl)
    res = []
    for i in range(3):
        for j in range(4):
            acc = b[0][i] * p0gt[(0 * 4 + j) * 16:(0 * 4 + j + 1) * 16, :]
            acc += b[1][i] * p0gt[(1 * 4 + j) * 16:(1 * 4 + j + 1) * 16, :]
            acc += b[2][i] * p0gt[(2 * 4 + j) * 16:(2 * 4 + j + 1) * 16, :]
            if j == 3:
                acc += sl[i]
            res.append(acc)
    arr = jnp.stack(res, axis=1).reshape(192, L)
    out_ref[:, :] = jnp.transpose(arr)


def kernel(rots_l0, transls_l0, motion_coefs_l0, rots_l1, transls_l1,
           motion_coefs_l1, parent_indices_l0, parent_indices_l1, ts):
    N0, B, Fr = rots_l1.shape[0], rots_l1.shape[1], rots_l1.shape[2]
    T = ts.shape[0]
    N1 = motion_coefs_l1.shape[0]

    d1 = jnp.concatenate([transls_l1, rots_l1], axis=-1)
    d1 = jnp.transpose(d1, (3, 2, 0, 1)).reshape(9, Fr, N0 * B)
    d1 = jnp.pad(d1, ((0, 0), (0, _F_PAD - Fr), (0, 0)))
    d0 = jnp.concatenate([transls_l0, rots_l0], axis=-1)[0]
    d0 = jnp.transpose(d0, (2, 1, 0))
    d0 = jnp.pad(d0, ((0, 0), (0, _F_PAD - Fr), (0, 0)))
    ts2 = ts.reshape(T, 1).astype(jnp.int32)
    mc0t = motion_coefs_l0.T

    mallt, p0t = pl.pallas_call(
        _prep_kernel,
        out_shape=[
            jax.ShapeDtypeStruct((9 * 16, N0 * B), jnp.float32),
            jax.ShapeDtypeStruct((12 * 16, N0), jnp.float32),
        ],
    )(ts2, d1, d0, mc0t)

    coefst = motion_coefs_l1.T
    part = parent_indices_l1.astype(jnp.int32).reshape(1, N1)
    grid = (N1 + _TILE - 1) // _TILE
    out = pl.pallas_call(
        _main_kernel,
        grid=(grid,),
        in_specs=[
            pl.BlockSpec((16, _TILE), lambda i: (0, i)),
            pl.BlockSpec((1, _TILE), lambda i: (0, i)),
            pl.BlockSpec((9 * 16, N0 * B), lambda i: (0, 0)),
            pl.BlockSpec((12 * 16, N0), lambda i: (0, 0)),
        ],
        out_specs=pl.BlockSpec((_TILE, 12 * 16), lambda i: (i, 0)),
        out_shape=jax.ShapeDtypeStruct((N1, 12 * 16), jnp.float32),
    )(coefst, part, mallt, p0t)

    return out.reshape(N1, T, 3, 4)
